# baseline (speedup 1.0000x reference)
import functools

import jax
import jax.numpy as jnp
from jax.experimental import pallas as pl
from jax.experimental.pallas import tpu as pltpu

TOPK = 2
E = 16
BLOCK_M = 1024


def _router_kernel(x_ref, wt_ref, rw_ref, sel_ref, aux_ref, acc_ref,
                   *, nblocks, inv_total):
    i = pl.program_id(0)
    x = x_ref[...]
    wt = wt_ref[...]
    logits = jnp.dot(x, wt, preferred_element_type=jnp.float32)

    m1 = jnp.max(logits, axis=1, keepdims=True)
    i1 = jnp.argmax(logits, axis=1)
    eidx = jax.lax.broadcasted_iota(jnp.int32, logits.shape, 1)
    masked = jnp.where(eidx == i1[:, None], -jnp.inf, logits)
    m2 = jnp.max(masked, axis=1, keepdims=True)
    i2 = jnp.argmax(masked, axis=1)

    e2 = jnp.exp(m2 - m1)
    denom = 1.0 + e2
    w1 = 1.0 / denom
    w2 = e2 / denom
    rw_ref[...] = jnp.concatenate([w1, w2], axis=1)
    sel_ref[...] = jnp.concatenate([i1[:, None], i2[:, None]], axis=1)

    p = jnp.exp(logits - m1)
    p = p / jnp.sum(p, axis=1, keepdims=True)
    psum = jnp.sum(p, axis=0, keepdims=True)

    @pl.when(i == 0)
    def _():
        acc_ref[...] = jnp.zeros_like(acc_ref)

    acc_ref[...] += psum

    @pl.when(i == nblocks - 1)
    def _():
        mean_pe = acc_ref[...] * inv_total
        aux_ref[...] = E * jnp.sum(mean_pe * mean_pe, axis=(0, 1),
                                   keepdims=True)


def kernel(hidden_states, gate_weight):
    b, s, h = hidden_states.shape
    n = b * s
    x = hidden_states.reshape(n, h)
    wt = gate_weight.T
    nblocks = n // BLOCK_M

    body = functools.partial(_router_kernel, nblocks=nblocks,
                             inv_total=1.0 / n)
    rw, sel, aux = pl.pallas_call(
        body,
        grid=(nblocks,),
        in_specs=[
            pl.BlockSpec((BLOCK_M, h), lambda i: (i, 0)),
            pl.BlockSpec((h, E), lambda i: (0, 0)),
        ],
        out_specs=[
            pl.BlockSpec((BLOCK_M, TOPK), lambda i: (i, 0)),
            pl.BlockSpec((BLOCK_M, TOPK), lambda i: (i, 0)),
            pl.BlockSpec((1, 1), lambda i: (0, 0)),
        ],
        out_shape=[
            jax.ShapeDtypeStruct((n, TOPK), jnp.float32),
            jax.ShapeDtypeStruct((n, TOPK), jnp.int32),
            jax.ShapeDtypeStruct((1, 1), jnp.float32),
        ],
        scratch_shapes=[pltpu.VMEM((1, E), jnp.float32)],
    )(x, wt)

    return (rw.reshape(b, s, TOPK), sel.reshape(b, s, TOPK), aux[0, 0])


# speedup vs baseline: 1.0238x; 1.0238x over previous
import functools

import jax
import jax.numpy as jnp
from jax.experimental import pallas as pl
from jax.experimental.pallas import tpu as pltpu

TOPK = 2
E = 16
BLOCK_M = 2048


def _router_kernel(x_ref, wt_ref, rw_ref, sel_ref, aux_ref, acc_ref,
                   *, nblocks, inv_total):
    i = pl.program_id(0)
    x = x_ref[...]
    wt = wt_ref[...]
    logits = jnp.dot(x, wt, preferred_element_type=jnp.float32)

    m1 = jnp.max(logits, axis=1, keepdims=True)
    i1 = jnp.argmax(logits, axis=1)
    eidx = jax.lax.broadcasted_iota(jnp.int32, logits.shape, 1)
    masked = jnp.where(eidx == i1[:, None], -jnp.inf, logits)
    m2 = jnp.max(masked, axis=1, keepdims=True)
    i2 = jnp.argmax(masked, axis=1)

    e2 = jnp.exp(m2 - m1)
    denom = 1.0 + e2
    w1 = 1.0 / denom
    w2 = e2 / denom
    rw_ref[...] = jnp.concatenate([w1, w2], axis=1)
    sel_ref[...] = jnp.concatenate([i1[:, None], i2[:, None]], axis=1)

    p = jnp.exp(logits - m1)
    p = p / jnp.sum(p, axis=1, keepdims=True)
    psum = jnp.sum(p, axis=0, keepdims=True)

    @pl.when(i == 0)
    def _():
        acc_ref[...] = jnp.zeros_like(acc_ref)

    acc_ref[...] += psum

    @pl.when(i == nblocks - 1)
    def _():
        mean_pe = acc_ref[...] * inv_total
        aux_ref[...] = E * jnp.sum(mean_pe * mean_pe, axis=(0, 1),
                                   keepdims=True)


def kernel(hidden_states, gate_weight):
    b, s, h = hidden_states.shape
    n = b * s
    x = hidden_states.reshape(n, h)
    wt = gate_weight.T
    nblocks = n // BLOCK_M

    body = functools.partial(_router_kernel, nblocks=nblocks,
                             inv_total=1.0 / n)
    rw, sel, aux = pl.pallas_call(
        body,
        grid=(nblocks,),
        in_specs=[
            pl.BlockSpec((BLOCK_M, h), lambda i: (i, 0)),
            pl.BlockSpec((h, E), lambda i: (0, 0)),
        ],
        out_specs=[
            pl.BlockSpec((BLOCK_M, TOPK), lambda i: (i, 0)),
            pl.BlockSpec((BLOCK_M, TOPK), lambda i: (i, 0)),
            pl.BlockSpec((1, 1), lambda i: (0, 0)),
        ],
        out_shape=[
            jax.ShapeDtypeStruct((n, TOPK), jnp.float32),
            jax.ShapeDtypeStruct((n, TOPK), jnp.int32),
            jax.ShapeDtypeStruct((1, 1), jnp.float32),
        ],
        scratch_shapes=[pltpu.VMEM((1, E), jnp.float32)],
    )(x, wt)

    return (rw.reshape(b, s, TOPK), sel.reshape(b, s, TOPK), aux[0, 0])
